# baseline (speedup 1.0000x reference)
import functools

import jax
import jax.numpy as jnp
from jax import lax
from jax.experimental import pallas as pl
from jax.experimental.pallas import tpu as pltpu
from jax.experimental.pallas import tpu_sc as plsc

_B, _S, _H, _V = 4, 8192, 1024, 100000
_NW = 32
_TOK = _B * _S
_SPW = _S // _NW
_P = 8
_NG = _SPW // _P
_GR = _B * _P
_VPR = _H // 16
_NBUF = 3


def _add_pos_half(rows_ref, pos_ref, half):
    _G = 8

    def row_body(r, c):
        for g in range(_VPR // _G):
            vals = [pos_ref[r, pl.ds((g * _G + u) * 16, 16)] for u in range(_G)]
            for b in (2 * half, 2 * half + 1):
                for u in range(_G):
                    plsc.addupdate(
                        rows_ref.at[b * _P + r, pl.ds((g * _G + u) * 16, 16)],
                        vals[u],
                    )
        return c

    lax.fori_loop(0, _P, row_body, 0)


def _emb_kernel(ids_hbm, pt_hbm, wt_hbm, out_hbm,
                idx_lin, pos0, pos1, rows0, rows1, rows2,
                g0a, g0b, g1a, g1b, g2a, g2b, s0, s1, s2, p0, p1):
    wid = lax.axis_index("s") * 2 + lax.axis_index("c")
    s_base = wid * _SPW
    rows = (rows0, rows1, rows2)
    gsem = ((g0a, g0b), (g1a, g1b), (g2a, g2b))
    ssem = (s0, s1, s2)
    pos = (pos0, pos1)
    psem = (p0, p1)

    idx_cp = pltpu.async_copy(
        ids_hbm.at[pl.ds(wid * (_B * _SPW), _B * _SPW)], idx_lin, p1
    )

    _HR = _GR // 2

    def gather_half(g, q, half):
        return (wt_hbm.at[idx_lin.at[pl.ds(g * _GR + half * _HR, _HR)]],
                rows[q].at[pl.ds(half * _HR, _HR)], gsem[q][half])

    def issue_gather(g, q):
        for half in (0, 1):
            src, dst, sem = gather_half(g, q, half)
            pltpu.async_copy(src, dst, sem)

    def wait_gather_half(g, q, half):
        src, dst, sem = gather_half(g, q, half)
        pltpu.make_async_copy(src, dst, sem).wait()

    def store_pair(g, b, q):
        return (rows[q].at[pl.ds(b * _P, _P)],
                out_hbm.at[pl.ds(b * _S + s_base + g * _P, _P)])

    def pos_src(g):
        return pt_hbm.at[pl.ds(s_base + g * _P, _P)]

    pltpu.async_copy(pos_src(0), pos0, p0)
    idx_cp.wait()
    issue_gather(0, 0)
    issue_gather(1, 1)

    def body(g, q, pp):
        rbuf = rows[q]
        @pl.when(g + 1 < _NG)
        def _():
            pltpu.async_copy(pos_src(g + 1), pos[1 - pp], psem[1 - pp])

        qn = (q + 2) % _NBUF

        @pl.when(g >= 1)
        def _():
            for b in range(_B):
                src, dst = store_pair(g - 1, b, qn)
                pltpu.make_async_copy(src, dst, ssem[qn]).wait()

        @pl.when(g + 2 < _NG)
        def _():
            issue_gather(g + 2, qn)

        wait_gather_half(g, q, 0)
        pltpu.make_async_copy(pos_src(g), pos[pp], psem[pp]).wait()
        _add_pos_half(rbuf, pos[pp], 0)
        wait_gather_half(g, q, 1)
        _add_pos_half(rbuf, pos[pp], 1)
        for b in range(_B):
            src, dst = store_pair(g, b, q)
            pltpu.async_copy(src, dst, ssem[q])

    def six_body(h, carry):
        for u in range(6):
            body(6 * h + u, u % _NBUF, u % 2)
        return carry

    lax.fori_loop(0, _NG // 6, six_body, 0)
    for g in range(_NG - (_NG % 6), _NG):
        body(g, g % _NBUF, g % 2)
    gl = _NG - 1
    for b in range(_B):
        src, dst = store_pair(gl, b, gl % _NBUF)
        pltpu.make_async_copy(src, dst, ssem[gl % _NBUF]).wait()


@jax.jit
def _run(ids_flat, word_table, pos_table):
    mesh = plsc.VectorSubcoreMesh(core_axis_name="c", subcore_axis_name="s")
    k = functools.partial(
        pl.kernel,
        mesh=mesh,
        out_type=jax.ShapeDtypeStruct((_TOK, _H), jnp.float32),
        scratch_types=[
            pltpu.VMEM((_B * _SPW,), jnp.int32),
            pltpu.VMEM((_P, _H), jnp.float32),
            pltpu.VMEM((_P, _H), jnp.float32),
        ] + [pltpu.VMEM((_GR, _H), jnp.float32)] * _NBUF
          + [pltpu.SemaphoreType.DMA] * (3 * _NBUF + 2),
    )(_emb_kernel)
    return k(ids_flat, pos_table, word_table)


def kernel(input_ids, word_table, pos_table):
    ids_g = jnp.transpose(
        input_ids.astype(jnp.int32).reshape(_B, _NW, _NG, _P), (1, 2, 0, 3)
    ).reshape(_TOK)
    out = _run(ids_g, word_table, pos_table)
    return out.reshape(_B, _S, _H)


# speedup vs baseline: 1.0260x; 1.0260x over previous
import functools

import jax
import jax.numpy as jnp
from jax import lax
from jax.experimental import pallas as pl
from jax.experimental.pallas import tpu as pltpu
from jax.experimental.pallas import tpu_sc as plsc

_B, _S, _H, _V = 4, 8192, 1024, 100000
_NW = 32
_TOK = _B * _S
_SPW = _S // _NW
_P = 8
_NG = _SPW // _P
_GR = _B * _P
_VPR = _H // 16
_NBUF = 3


def _add_pos(rows_ref, pos_ref):
    _G = 16

    def row_body(r, c):
        for g in range(_VPR // _G):
            vals = [pos_ref[r, pl.ds((g * _G + u) * 16, 16)] for u in range(_G)]
            for b in range(_B):
                for u in range(_G):
                    plsc.addupdate(
                        rows_ref.at[b * _P + r, pl.ds((g * _G + u) * 16, 16)],
                        vals[u],
                    )
        return c

    lax.fori_loop(0, _P, row_body, 0)


def _emb_kernel(ids_hbm, pt_hbm, wt_hbm, out_hbm,
                idx_lin, pos0, pos1, rows0, rows1, rows2,
                g0, g1, g2, s0, s1, s2, p0, p1):
    wid = lax.axis_index("s") * 2 + lax.axis_index("c")
    s_base = wid * _SPW
    rows = (rows0, rows1, rows2)
    gsem = (g0, g1, g2)
    ssem = (s0, s1, s2)
    pos = (pos0, pos1)
    psem = (p0, p1)

    idx_cp = pltpu.async_copy(
        ids_hbm.at[pl.ds(wid * (_B * _SPW), _B * _SPW)], idx_lin, p1
    )

    def issue_gather(g, q):
        pltpu.async_copy(
            wt_hbm.at[idx_lin.at[pl.ds(g * _GR, _GR)]], rows[q], gsem[q]
        )

    def wait_gather(g, q):
        pltpu.make_async_copy(
            wt_hbm.at[idx_lin.at[pl.ds(g * _GR, _GR)]], rows[q], gsem[q]
        ).wait()

    def store_pair(g, b, q):
        return (rows[q].at[pl.ds(b * _P, _P)],
                out_hbm.at[pl.ds(b * _S + s_base + g * _P, _P)])

    def pos_src(g):
        return pt_hbm.at[pl.ds(s_base + g * _P, _P)]

    pltpu.async_copy(pos_src(0), pos0, p0)
    idx_cp.wait()
    issue_gather(0, 0)
    issue_gather(1, 1)

    def body(g, q, pp):
        rbuf = rows[q]
        wait_gather(g, q)
        @pl.when(g + 1 < _NG)
        def _():
            pltpu.async_copy(pos_src(g + 1), pos[1 - pp], psem[1 - pp])

        qn = (q + 2) % _NBUF

        @pl.when(g >= 1)
        def _():
            for b in range(_B):
                src, dst = store_pair(g - 1, b, qn)
                pltpu.make_async_copy(src, dst, ssem[qn]).wait()

        @pl.when(g + 2 < _NG)
        def _():
            issue_gather(g + 2, qn)

        pltpu.make_async_copy(pos_src(g), pos[pp], psem[pp]).wait()
        _add_pos(rbuf, pos[pp])
        for b in range(_B):
            src, dst = store_pair(g, b, q)
            pltpu.async_copy(src, dst, ssem[q])

    def six_body(h, carry):
        for u in range(6):
            body(6 * h + u, u % _NBUF, u % 2)
        return carry

    lax.fori_loop(0, _NG // 6, six_body, 0)
    for g in range(_NG - (_NG % 6), _NG):
        body(g, g % _NBUF, g % 2)
    gl = _NG - 1
    for b in range(_B):
        src, dst = store_pair(gl, b, gl % _NBUF)
        pltpu.make_async_copy(src, dst, ssem[gl % _NBUF]).wait()


@jax.jit
def _run(ids_flat, word_table, pos_table):
    mesh = plsc.VectorSubcoreMesh(core_axis_name="c", subcore_axis_name="s")
    k = functools.partial(
        pl.kernel,
        mesh=mesh,
        out_type=jax.ShapeDtypeStruct((_TOK, _H), jnp.float32),
        scratch_types=[
            pltpu.VMEM((_B * _SPW,), jnp.int32),
            pltpu.VMEM((_P, _H), jnp.float32),
            pltpu.VMEM((_P, _H), jnp.float32),
        ] + [pltpu.VMEM((_GR, _H), jnp.float32)] * _NBUF
          + [pltpu.SemaphoreType.DMA] * (2 * _NBUF + 2),
    )(_emb_kernel)
    return k(ids_flat, pos_table, word_table)


def kernel(input_ids, word_table, pos_table):
    ids_g = jnp.transpose(
        input_ids.astype(jnp.int32).reshape(_B, _NW, _NG, _P), (1, 2, 0, 3)
    ).reshape(_TOK)
    out = _run(ids_g, word_table, pos_table)
    return out.reshape(_B, _S, _H)
